# baseline (speedup 1.0000x reference)
import functools

import jax
import jax.numpy as jnp
from jax import lax
from jax.experimental import pallas as pl
from jax.experimental.pallas import tpu as pltpu
from jax.experimental.pallas import tpu_sc as plsc

_C = 32
_M = 128
_NG = 16
_D = _C * _NG
_N = 100000
_CH = 80
_NCH = _N // _CH
_NW = 32
_LBW = 2048
_NLB = (_N + _LBW - 1) // _LBW
_NPAD = _NLB * _LBW


def _table_body(b_ref, pi_ref, tab_ref, llt_ref):
    b = b_ref[...]
    pi = pi_ref[...][:1, :]
    r0 = lax.broadcasted_iota(jnp.int32, (_D, _D), 0)
    r1 = lax.broadcasted_iota(jnp.int32, (_D, _D), 1)
    k_mat = ((r0 % _NG) == (r1 % _NG)).astype(jnp.float32)

    bmax = jnp.max(b, axis=0, keepdims=True)
    be = jnp.exp(b - bmax)
    sm_b = be / jnp.sum(be, axis=0, keepdims=True)

    pe = jnp.exp(pi)
    pz = jnp.dot(pe, k_mat, preferred_element_type=jnp.float32)
    sm_pi = pe / pz

    un = sm_pi * sm_b + 1e-8
    s_b = jnp.dot(un, k_mat, preferred_element_type=jnp.float32)
    tab_ref[...] = un / s_b
    llt_ref[...] = jnp.transpose(jnp.log(s_b[:, :_NG]), (1, 0))


def _build_tables(b2d, pi2d):
    return pl.pallas_call(
        _table_body,
        out_shape=(
            jax.ShapeDtypeStruct((_M, _D), jnp.float32),
            jax.ShapeDtypeStruct((_NG, _M), jnp.float32),
        ),
    )(b2d, pi2d)


def _make_gather_sync(dim, ch, dtype):
    nch = _N // ch
    tmin = nch // _NW
    tmax = tmin + 1
    xrem = nch % _NW

    def body(tab_hbm, x_hbm, post_out, idx_v, rows_v, rows_b,
             gsem, gsemb, wsem, wsemb):
        wid = lax.axis_index("s") * 2 + lax.axis_index("c")
        nt = tmin + (wid < xrem).astype(jnp.int32)
        nbase = (wid * tmin + jnp.minimum(wid, xrem)) * ch

        pltpu.sync_copy(x_hbm.at[pl.ds(nbase, tmin * ch)],
                        idx_v.at[pl.ds(0, tmin * ch)])

        @pl.when(wid < xrem)
        def _extra():
            pltpu.sync_copy(x_hbm.at[pl.ds(nbase + tmin * ch, ch)],
                            idx_v.at[pl.ds(tmin * ch, ch)])

        h = ch // 2

        def chunk(t, carry):
            ga = pltpu.async_copy(
                tab_hbm.at[idx_v.at[pl.ds(t * ch, h)]], rows_v, gsem)
            gb = pltpu.async_copy(
                tab_hbm.at[idx_v.at[pl.ds(t * ch + h, h)]], rows_b, gsemb)
            ga.wait()
            wa = pltpu.async_copy(
                rows_v, post_out.at[pl.ds(nbase + t * ch, h)], wsem)
            gb.wait()
            wb = pltpu.async_copy(
                rows_b, post_out.at[pl.ds(nbase + t * ch + h, h)], wsemb)
            wa.wait()
            wb.wait()
            return carry

        lax.fori_loop(0, nt, chunk, 0)

    mesh = plsc.VectorSubcoreMesh(core_axis_name="c", subcore_axis_name="s")
    return functools.partial(
        pl.kernel,
        mesh=mesh,
        out_type=jax.ShapeDtypeStruct((_N, dim), dtype),
        scratch_types=[
            pltpu.VMEM((tmax * ch,), jnp.int32),
            pltpu.VMEM((ch // 2, dim), dtype),
            pltpu.VMEM((ch // 2, dim), dtype),
            pltpu.SemaphoreType.DMA,
            pltpu.SemaphoreType.DMA,
            pltpu.SemaphoreType.DMA,
            pltpu.SemaphoreType.DMA,
        ],
    )(body)


def _make_gather(dim, ch, dtype):
    nch = _N // ch
    tmin = nch // _NW
    tmax = tmin + 1
    xrem = nch % _NW

    def body(tab_hbm, x_hbm, post_out, idx_v,
             rows0, rows1, gsem0, gsem1, wsem0, wsem1):
        wid = lax.axis_index("s") * 2 + lax.axis_index("c")
        nt = tmin + (wid < xrem).astype(jnp.int32)
        nbase = (wid * tmin + jnp.minimum(wid, xrem)) * ch

        pltpu.sync_copy(x_hbm.at[pl.ds(nbase, tmin * ch)],
                        idx_v.at[pl.ds(0, tmin * ch)])

        @pl.when(wid < xrem)
        def _extra():
            pltpu.sync_copy(x_hbm.at[pl.ds(nbase + tmin * ch, ch)],
                            idx_v.at[pl.ds(tmin * ch, ch)])

        rows = (rows0, rows1)
        gsems = (gsem0, gsem1)
        wsems = (wsem0, wsem1)
        pending = [None, None]

        for t in range(tmax):
            slot = t % 2

            if pending[slot] is not None:
                prev = pending[slot]

                @pl.when(t - 2 < nt)
                def _drain():
                    prev.wait()

            @pl.when(t < nt)
            def _work():
                pltpu.async_copy(
                    tab_hbm.at[idx_v.at[pl.ds(t * ch, ch)]],
                    rows[slot], gsems[slot]).wait()
                pltpu.async_copy(
                    rows[slot],
                    post_out.at[pl.ds(nbase + t * ch, ch)],
                    wsems[slot])

            pending[slot] = pltpu.make_async_copy(
                rows[slot],
                post_out.at[pl.ds(nbase + t * ch, ch)],
                wsems[slot])

        for t in (tmax - 2, tmax - 1):
            slot = t % 2
            prev = pending[slot]

            @pl.when(t < nt)
            def _drain2():
                prev.wait()

    mesh = plsc.VectorSubcoreMesh(core_axis_name="c", subcore_axis_name="s")
    return functools.partial(
        pl.kernel,
        mesh=mesh,
        out_type=jax.ShapeDtypeStruct((_N, dim), dtype),
        scratch_types=[
            pltpu.VMEM((tmax * ch,), jnp.int32),
            pltpu.VMEM((ch, dim), dtype),
            pltpu.VMEM((ch, dim), dtype),
            pltpu.SemaphoreType.DMA,
            pltpu.SemaphoreType.DMA,
            pltpu.SemaphoreType.DMA,
            pltpu.SemaphoreType.DMA,
        ],
    )(body)


_UBW = 1024
_NUB = (_N + _UBW - 1) // _UBW


def _unpack_body(w_ref, out_ref):
    wu = lax.bitcast_convert_type(w_ref[...], jnp.uint32)
    lo = lax.bitcast_convert_type(wu << 16, jnp.float32)
    hi = lax.bitcast_convert_type(wu & jnp.uint32(0xFFFF0000),
                                  jnp.float32)
    out_ref[...] = jnp.concatenate(
        [jnp.transpose(lo, (1, 0)), jnp.transpose(hi, (1, 0))], axis=0)


def _unpack(post_p):
    return pl.pallas_call(
        _unpack_body,
        grid=(_NUB,),
        in_specs=[pl.BlockSpec((_UBW, _D // 2), lambda i: (i, 0))],
        out_specs=pl.BlockSpec((_D, _UBW), lambda i: (0, i)),
        out_shape=jax.ShapeDtypeStruct((_D, _N), jnp.float32),
    )(post_p)


def _ll_body(llt_ref, x_ref, out_ref):
    i = pl.program_id(0)
    xs = x_ref[:, pl.ds(i * _LBW, _LBW)]
    m = lax.broadcasted_iota(jnp.int32, (_M, 1), 0)
    onehot = (xs == m).astype(jnp.float32)
    out_ref[...] = jnp.dot(llt_ref[...], onehot,
                           preferred_element_type=jnp.float32)


def _ll_matmul(llt, x2):
    return pl.pallas_call(
        _ll_body,
        grid=(_NLB,),
        in_specs=[
            pl.BlockSpec((_NG, _M), lambda i: (0, 0)),
            pl.BlockSpec((1, _NPAD), lambda i: (0, 0)),
        ],
        out_specs=pl.BlockSpec((_NG, _LBW), lambda i: (0, i)),
        out_shape=jax.ShapeDtypeStruct((_NG, _N), jnp.float32),
    )(llt, x2)


def kernel(x, B, Pi):
    xi = x.astype(jnp.int32)
    b2d = jnp.transpose(B, (1, 0, 2)).reshape(_M, _D)
    pi2d = jnp.broadcast_to(Pi.reshape(1, _D), (8, _D))
    tab, llt = _build_tables(b2d, pi2d)
    aw = lax.bitcast_convert_type(
        tab.astype(jnp.bfloat16), jnp.uint16).astype(jnp.uint32)
    tabp = lax.bitcast_convert_type(
        aw[:, : _D // 2] | (aw[:, _D // 2:] << 16), jnp.int32)
    post_p = _make_gather_sync(_D // 2, 160, jnp.int32)(tabp, xi)
    xp = jnp.pad(xi, (0, _NPAD - _N)).reshape(1, _NPAD)
    ll_t = _ll_matmul(llt, xp)
    post_t = _unpack(post_p)
    posterior = jnp.transpose(post_t.reshape(_C, _NG, _N), (2, 0, 1))
    return jnp.transpose(ll_t, (1, 0)), posterior


# speedup vs baseline: 1.1091x; 1.1091x over previous
import functools

import jax
import jax.numpy as jnp
from jax import lax
from jax.experimental import pallas as pl
from jax.experimental.pallas import tpu as pltpu
from jax.experimental.pallas import tpu_sc as plsc

_C = 32
_M = 128
_NG = 16
_D = _C * _NG
_N = 100000
_NW = 32
_LBW = 2048
_NLB = (_N + _LBW - 1) // _LBW
_NPAD = _NLB * _LBW


def _table_body(b_ref, pi_ref, tab_ref, llt_ref):
    b = b_ref[...]
    pi = pi_ref[...][:1, :]
    r0 = lax.broadcasted_iota(jnp.int32, (_D, _D), 0)
    r1 = lax.broadcasted_iota(jnp.int32, (_D, _D), 1)
    k_mat = ((r0 % _NG) == (r1 % _NG)).astype(jnp.float32)

    bmax = jnp.max(b, axis=0, keepdims=True)
    be = jnp.exp(b - bmax)
    sm_b = be / jnp.sum(be, axis=0, keepdims=True)

    pe = jnp.exp(pi)
    pz = jnp.dot(pe, k_mat, preferred_element_type=jnp.float32)
    sm_pi = pe / pz

    un = sm_pi * sm_b + 1e-8
    s_b = jnp.dot(un, k_mat, preferred_element_type=jnp.float32)
    tab_ref[...] = un / s_b
    llt_ref[...] = jnp.transpose(jnp.log(s_b[:, :_NG]), (1, 0))


def _build_tables(b2d, pi2d):
    return pl.pallas_call(
        _table_body,
        out_shape=(
            jax.ShapeDtypeStruct((_M, _D), jnp.float32),
            jax.ShapeDtypeStruct((_NG, _M), jnp.float32),
        ),
    )(b2d, pi2d)


def _make_gather_sync(dim, ch, dtype):
    nch = _N // ch
    tmin = nch // _NW
    tmax = tmin + 1
    xrem = nch % _NW

    def body(tab_hbm, x_hbm, post_out, idx_v, rows_v, gsem, wsem):
        wid = lax.axis_index("s") * 2 + lax.axis_index("c")
        nt = tmin + (wid < xrem).astype(jnp.int32)
        nbase = (wid * tmin + jnp.minimum(wid, xrem)) * ch

        pltpu.sync_copy(x_hbm.at[pl.ds(nbase, tmin * ch)],
                        idx_v.at[pl.ds(0, tmin * ch)])

        @pl.when(wid < xrem)
        def _extra():
            pltpu.sync_copy(x_hbm.at[pl.ds(nbase + tmin * ch, ch)],
                            idx_v.at[pl.ds(tmin * ch, ch)])

        def chunk(t, carry):
            pltpu.async_copy(
                tab_hbm.at[idx_v.at[pl.ds(t * ch, ch)]],
                rows_v, gsem).wait()
            pltpu.async_copy(
                rows_v, post_out.at[pl.ds(nbase + t * ch, ch)], wsem).wait()
            return carry

        lax.fori_loop(0, nt, chunk, 0)

    mesh = plsc.VectorSubcoreMesh(core_axis_name="c", subcore_axis_name="s")
    return functools.partial(
        pl.kernel,
        mesh=mesh,
        out_type=jax.ShapeDtypeStruct((_N, dim), dtype),
        scratch_types=[
            pltpu.VMEM((tmax * ch,), jnp.int32),
            pltpu.VMEM((ch, dim), dtype),
            pltpu.SemaphoreType.DMA,
            pltpu.SemaphoreType.DMA,
        ],
    )(body)


_UBW = 8192
_NUB = (_N + _UBW - 1) // _UBW


def _unpack_body(w_ref, out_ref):
    wu = lax.bitcast_convert_type(w_ref[...], jnp.uint32)
    lo = lax.bitcast_convert_type(wu << 16, jnp.float32)
    hi = lax.bitcast_convert_type(wu & jnp.uint32(0xFFFF0000),
                                  jnp.float32)
    out_ref[...] = jnp.concatenate(
        [jnp.transpose(lo, (1, 0)), jnp.transpose(hi, (1, 0))], axis=0)


def _unpack(post_p):
    return pl.pallas_call(
        _unpack_body,
        grid=(_NUB,),
        in_specs=[pl.BlockSpec((_UBW, _D // 2), lambda i: (i, 0))],
        out_specs=pl.BlockSpec((_D, _UBW), lambda i: (0, i)),
        out_shape=jax.ShapeDtypeStruct((_D, _N), jnp.float32),
    )(post_p)


def _ll_body(llt_ref, x_ref, out_ref):
    i = pl.program_id(0)
    xs = x_ref[:, pl.ds(i * _LBW, _LBW)]
    m = lax.broadcasted_iota(jnp.int32, (_M, 1), 0)
    onehot = (xs == m).astype(jnp.float32)
    out_ref[...] = jnp.dot(llt_ref[...], onehot,
                           preferred_element_type=jnp.float32)


def _ll_matmul(llt, x2):
    return pl.pallas_call(
        _ll_body,
        grid=(_NLB,),
        in_specs=[
            pl.BlockSpec((_NG, _M), lambda i: (0, 0)),
            pl.BlockSpec((1, _NPAD), lambda i: (0, 0)),
        ],
        out_specs=pl.BlockSpec((_NG, _LBW), lambda i: (0, i)),
        out_shape=jax.ShapeDtypeStruct((_NG, _N), jnp.float32),
    )(llt, x2)


def kernel(x, B, Pi):
    xi = x.astype(jnp.int32)
    b2d = jnp.transpose(B, (1, 0, 2)).reshape(_M, _D)
    pi2d = jnp.broadcast_to(Pi.reshape(1, _D), (8, _D))
    tab, llt = _build_tables(b2d, pi2d)
    aw = lax.bitcast_convert_type(
        tab.astype(jnp.bfloat16), jnp.uint16).astype(jnp.uint32)
    tabp = lax.bitcast_convert_type(
        aw[:, : _D // 2] | (aw[:, _D // 2:] << 16), jnp.int32)
    post_p = _make_gather_sync(_D // 2, 200, jnp.int32)(tabp, xi)
    xp = jnp.pad(xi, (0, _NPAD - _N)).reshape(1, _NPAD)
    ll_t = _ll_matmul(llt, xp)
    post_t = _unpack(post_p)
    posterior = jnp.transpose(post_t.reshape(_C, _NG, _N), (2, 0, 1))
    return jnp.transpose(ll_t, (1, 0)), posterior
